# baseline (speedup 1.0000x reference)
import functools

import jax
import jax.numpy as jnp
from jax import lax
from jax.experimental import pallas as pl
from jax.experimental.pallas import tpu as pltpu
from jax.experimental.pallas import tpu_sc as plsc

N_FIELDS = 26
VOCAB = 100000
EMBED_DIM = 16
N_DENSE = 13
BATCH = 16384
FPAD = 32
NSLAB = 4
SLAB_ROWS = BATCH * 8
ROWS = BATCH * FPAD

_NC = 2
_NS = 16
_NW = _NC * _NS
_RPW = ROWS // _NW
_CHR = 2048
_NCH = _RPW // _CHR


def _sc_gather(tables, idx_flat, offs):
    mesh = plsc.VectorSubcoreMesh(core_axis_name="c", subcore_axis_name="s")

    @functools.partial(
        pl.kernel,
        mesh=mesh,
        compiler_params=pltpu.CompilerParams(use_tc_tiling_on_sc=False),
        out_type=jax.ShapeDtypeStruct((ROWS, EMBED_DIM), jnp.float32),
        scratch_types=[
            pltpu.VMEM((_CHR,), jnp.int32),
            pltpu.VMEM((_CHR,), jnp.int32),
            pltpu.VMEM((_CHR, EMBED_DIM), jnp.float32),
            pltpu.VMEM((NSLAB * 16,), jnp.int32),
            pltpu.SemaphoreType.DMA,
        ],
    )
    def k(tab2, idx_hbm, off_hbm, out_hbm, raw_v, idx_v, rows_v, off_v, sem):
        wid = lax.axis_index("s") * _NC + lax.axis_index("c")
        pltpu.sync_copy(off_hbm, off_v)
        r0_w = wid * _RPW
        slab = wid // 8

        def chunk(ci, carry):
            r0 = r0_w + ci * _CHR
            pltpu.sync_copy(idx_hbm.at[pl.ds(r0, _CHR)], raw_v)
            off16 = off_v[pl.ds(slab * 16, 16)]

            def grp(m, carry2):
                sl = pl.ds(m * 16, 16)
                idx_v[sl] = raw_v[sl] + off16
                return carry2

            lax.fori_loop(0, _CHR // 16, grp, 0)
            pltpu.async_copy(tab2.at[idx_v], rows_v, sem).wait()
            pltpu.sync_copy(rows_v, out_hbm.at[pl.ds(r0, _CHR)])
            return carry

        lax.fori_loop(0, _NCH, chunk, 0)

    return k(tables, idx_flat, offs)


_BB = 1024


def _mlp_body(dense_ref, e0_ref, e1_ref, e2_ref, e3_ref, w1d_ref, w1e_ref,
              b1_ref, w2_ref, b2_ref, w3_ref, b3_ref, w4r_ref, wwr_ref,
              bsum_ref, out_ref):
    dense = dense_ref[...]
    h = jnp.dot(dense, w1d_ref[...], preferred_element_type=jnp.float32)
    for hh, eref in enumerate((e0_ref, e1_ref, e2_ref, e3_ref)):
        h = h + jnp.dot(eref[...], w1e_ref[hh * 128:(hh + 1) * 128, :],
                        preferred_element_type=jnp.float32)
    h = jnp.maximum(h + b1_ref[...], 0.0)
    h = jnp.maximum(jnp.dot(h, w2_ref[...], preferred_element_type=jnp.float32)
                    + b2_ref[...], 0.0)
    h = jnp.maximum(jnp.dot(h, w3_ref[...], preferred_element_type=jnp.float32)
                    + b3_ref[...], 0.0)
    deep = jnp.sum(h * w4r_ref[...], axis=1, keepdims=True)
    wide = jnp.sum(dense * wwr_ref[...], axis=1, keepdims=True)
    z = 0.5 * (wide + deep) + bsum_ref[...]
    out_ref[...] = 1.0 / (1.0 + jnp.exp(-z))


def _mlp(dense, emb128, w1d, w1e, b1, W2, b2, W3, b3, w4r, wwr, bsum):
    grid = (BATCH // _BB,)
    full = lambda shape: pl.BlockSpec(shape, lambda i: (0, 0))
    nblk = _BB // 128

    def slab_spec(hh):
        return pl.BlockSpec((_BB, 128),
                            lambda i, hh=hh: (hh * (BATCH // _BB) + i, 0))

    return pl.pallas_call(
        _mlp_body,
        grid=grid,
        in_specs=[
            pl.BlockSpec((_BB, N_DENSE), lambda i: (i, 0)),
            slab_spec(0), slab_spec(1), slab_spec(2), slab_spec(3),
            full(w1d.shape), full(w1e.shape), full(b1.shape),
            full(W2.shape), full(b2.shape),
            full(W3.shape), full(b3.shape),
            full(w4r.shape), full(wwr.shape), full(bsum.shape),
        ],
        out_specs=pl.BlockSpec((_BB, 1), lambda i: (i, 0)),
        out_shape=jax.ShapeDtypeStruct((BATCH, 1), jnp.float32),
    )(dense, emb128, emb128, emb128, emb128, w1d, w1e, b1, W2, b2, W3, b3,
      w4r, wwr, bsum)


def kernel(dense_inputs, sparse_inputs, tables, w_wide, b_wide,
           W1, b1, W2, b2, W3, b3, W4, b4):
    sp = sparse_inputs.astype(jnp.int32)
    sp_pad = jnp.pad(sp, ((0, 0), (0, FPAD - N_FIELDS)))
    idx_flat = (sp_pad.reshape(BATCH, NSLAB, 8)
                .transpose(1, 0, 2).reshape(ROWS))

    f_of = jnp.arange(NSLAB * 16, dtype=jnp.int32)
    f_val = (f_of // 16) * 8 + (f_of % 8)
    offs = jnp.where(f_val < N_FIELDS, f_val * VOCAB, 0)

    rows = _sc_gather(tables.reshape(N_FIELDS * VOCAB, EMBED_DIM),
                      idx_flat, offs)
    emb128 = rows.reshape(NSLAB * BATCH, 8 * EMBED_DIM)

    j = jnp.arange(NSLAB * 128)
    f = (j // 128) * 8 + (j % 128) // EMBED_DIM
    d = j % EMBED_DIM
    src = jnp.where(f < N_FIELDS, N_DENSE + f * EMBED_DIM + d, 0)
    w1e = jnp.where((f < N_FIELDS)[:, None], W1[src], 0.0)

    w1d = W1[:N_DENSE]
    w4r = W4.reshape(1, -1)
    wwr = w_wide.reshape(1, -1)
    bsum = (0.5 * (b_wide + b4)).reshape(1, 1)

    return _mlp(dense_inputs, emb128, w1d, w1e, b1.reshape(1, -1),
                W2, b2.reshape(1, -1), W3, b3.reshape(1, -1), w4r, wwr, bsum)


# speedup vs baseline: 1.3934x; 1.3934x over previous
import functools

import jax
import jax.numpy as jnp
from jax import lax
from jax.experimental import pallas as pl
from jax.experimental.pallas import tpu as pltpu
from jax.experimental.pallas import tpu_sc as plsc

N_FIELDS = 26
VOCAB = 100000
EMBED_DIM = 16
N_DENSE = 13
BATCH = 16384
FPAD = 32
NSLAB = 4
SLAB_ROWS = BATCH * 8
ROWS = BATCH * FPAD

_NC = 2
_NS = 16
_NW = _NC * _NS
_RPW = ROWS // _NW
_CHR = 2048
_NCH = _RPW // _CHR


def _sc_gather(tables, idx_flat, offs):
    mesh = plsc.VectorSubcoreMesh(core_axis_name="c", subcore_axis_name="s")

    @functools.partial(
        pl.kernel,
        mesh=mesh,
        compiler_params=pltpu.CompilerParams(use_tc_tiling_on_sc=False),
        out_type=jax.ShapeDtypeStruct((ROWS // 8, 8 * EMBED_DIM), jnp.float32),
        scratch_types=[
            pltpu.VMEM((_CHR,), jnp.int32),
            pltpu.VMEM((_CHR,), jnp.int32),
            pltpu.VMEM((_CHR, EMBED_DIM), jnp.float32),
            pltpu.VMEM((_CHR // 8, 8 * EMBED_DIM), jnp.float32),
            pltpu.VMEM((NSLAB * 16,), jnp.int32),
            pltpu.SemaphoreType.DMA,
        ],
    )
    def k(tab2, idx_hbm, off_hbm, out_hbm, raw_v, idx_v, rows_v, stage_v,
          off_v, sem):
        wid = lax.axis_index("s") * _NC + lax.axis_index("c")
        pltpu.sync_copy(off_hbm, off_v)
        r0_w = wid * _RPW
        slab = wid // 8
        off16 = off_v[pl.ds(slab * 16, 16)]

        def chunk(ci, carry):
            r0 = r0_w + ci * _CHR
            pltpu.sync_copy(idx_hbm.at[pl.ds(r0, _CHR)], raw_v)

            def grp(m, carry2):
                s = m * 128
                for j in range(8):
                    sl = pl.ds(s + j * 16, 16)
                    idx_v[sl] = raw_v[sl] + off16
                return carry2

            lax.fori_loop(0, _CHR // 128, grp, 0)
            pltpu.async_copy(tab2.at[idx_v], rows_v, sem).wait()

            def row_loop(r2, carry3):
                for u in range(8):
                    stage_v[r2, pl.ds(u * EMBED_DIM, EMBED_DIM)] = (
                        rows_v[r2 * 8 + u])
                return carry3

            lax.fori_loop(0, _CHR // 8, row_loop, 0)
            pltpu.sync_copy(stage_v, out_hbm.at[pl.ds(r0 // 8, _CHR // 8)])
            return carry

        lax.fori_loop(0, _NCH, chunk, 0)

    return k(tables, idx_flat, offs)


_BB = 1024


def _mlp_body(dense_ref, e0_ref, e1_ref, e2_ref, e3_ref, w1d_ref, w1e_ref,
              b1_ref, w2_ref, b2_ref, w3_ref, b3_ref, w4r_ref, wwr_ref,
              bsum_ref, out_ref):
    dense = dense_ref[...]
    h = jnp.dot(dense, w1d_ref[...], preferred_element_type=jnp.float32)
    for hh, eref in enumerate((e0_ref, e1_ref, e2_ref, e3_ref)):
        h = h + jnp.dot(eref[...], w1e_ref[hh * 128:(hh + 1) * 128, :],
                        preferred_element_type=jnp.float32)
    h = jnp.maximum(h + b1_ref[...], 0.0)
    h = jnp.maximum(jnp.dot(h, w2_ref[...], preferred_element_type=jnp.float32)
                    + b2_ref[...], 0.0)
    h = jnp.maximum(jnp.dot(h, w3_ref[...], preferred_element_type=jnp.float32)
                    + b3_ref[...], 0.0)
    deep = jnp.sum(h * w4r_ref[...], axis=1, keepdims=True)
    wide = jnp.sum(dense * wwr_ref[...], axis=1, keepdims=True)
    z = 0.5 * (wide + deep) + bsum_ref[...]
    out_ref[...] = 1.0 / (1.0 + jnp.exp(-z))


def _mlp(dense, emb128, w1d, w1e, b1, W2, b2, W3, b3, w4r, wwr, bsum):
    grid = (BATCH // _BB,)
    full = lambda shape: pl.BlockSpec(shape, lambda i: (0, 0))
    nblk = _BB // 128

    def slab_spec(hh):
        return pl.BlockSpec((_BB, 128),
                            lambda i, hh=hh: (hh * (BATCH // _BB) + i, 0))

    return pl.pallas_call(
        _mlp_body,
        grid=grid,
        in_specs=[
            pl.BlockSpec((_BB, N_DENSE), lambda i: (i, 0)),
            slab_spec(0), slab_spec(1), slab_spec(2), slab_spec(3),
            full(w1d.shape), full(w1e.shape), full(b1.shape),
            full(W2.shape), full(b2.shape),
            full(W3.shape), full(b3.shape),
            full(w4r.shape), full(wwr.shape), full(bsum.shape),
        ],
        out_specs=pl.BlockSpec((_BB, 1), lambda i: (i, 0)),
        out_shape=jax.ShapeDtypeStruct((BATCH, 1), jnp.float32),
    )(dense, emb128, emb128, emb128, emb128, w1d, w1e, b1, W2, b2, W3, b3,
      w4r, wwr, bsum)


def kernel(dense_inputs, sparse_inputs, tables, w_wide, b_wide,
           W1, b1, W2, b2, W3, b3, W4, b4):
    sp = sparse_inputs.astype(jnp.int32)
    sp_pad = jnp.concatenate([sp, sp[:, :FPAD - N_FIELDS]], axis=1)
    idx_flat = (sp_pad.reshape(BATCH, NSLAB, 8)
                .transpose(1, 0, 2).reshape(ROWS))

    f_of = jnp.arange(NSLAB * 16, dtype=jnp.int32)
    f_val = (f_of // 16) * 8 + (f_of % 8)
    offs = jnp.where(f_val < N_FIELDS, f_val * VOCAB, 0)

    emb128 = _sc_gather(tables.reshape(N_FIELDS * VOCAB, EMBED_DIM),
                        idx_flat, offs)

    j = jnp.arange(NSLAB * 128)
    f = (j // 128) * 8 + (j % 128) // EMBED_DIM
    d = j % EMBED_DIM
    src = jnp.where(f < N_FIELDS, N_DENSE + f * EMBED_DIM + d, 0)
    w1e = jnp.where((f < N_FIELDS)[:, None], W1[src], 0.0)

    w1d = W1[:N_DENSE]
    w4r = W4.reshape(1, -1)
    wwr = w_wide.reshape(1, -1)
    bsum = (0.5 * (b_wide + b4)).reshape(1, 1)

    return _mlp(dense_inputs, emb128, w1d, w1e, b1.reshape(1, -1),
                W2, b2.reshape(1, -1), W3, b3.reshape(1, -1), w4r, wwr, bsum)
